# baseline (speedup 1.0000x reference)
import functools

import jax
import jax.numpy as jnp
from jax import lax
from jax.experimental import pallas as pl
from jax.experimental.pallas import tpu as pltpu
from jax.experimental.pallas import tpu_sc as plsc

_L = 16


@functools.lru_cache(maxsize=None)
def _build(B, T, D):
    info = plsc.get_sparse_core_info()
    NC, NS = info.num_cores, info.num_subcores
    NW = NC * NS
    assert B % (8 * NW) == 0 and D % _L == 0
    bpw = B // NW

    mesh = plsc.VectorSubcoreMesh(core_axis_name="c", subcore_axis_name="s")

    @functools.partial(
        pl.kernel,
        mesh=mesh,
        out_type=jax.ShapeDtypeStruct((B, D), jnp.float32),
        scratch_types=[
            pltpu.VMEM((bpw,), jnp.int32),
            pltpu.VMEM((bpw,), jnp.int32),
            pltpu.VMEM((bpw, D), jnp.float32),
            pltpu.SemaphoreType.DMA,
        ],
    )
    def fused_gather(x_hbm, len_hbm, out_hbm, len_v, idx_v, rows_v, sem):
        wid = lax.axis_index("s") * NC + lax.axis_index("c")
        base = wid * bpw
        pltpu.sync_copy(len_hbm.at[pl.ds(base, bpw)], len_v)
        lane = lax.iota(jnp.int32, _L)
        for i in range(bpw // _L):
            ln = len_v[pl.ds(i * _L, _L)]
            t = jnp.where(ln == jnp.int32(0), jnp.int32(T - 1), ln - 1)
            row = (base + i * _L) * T + lane * T + t
            idx_v[pl.ds(i * _L, _L)] = row
        pltpu.async_copy(x_hbm.at[idx_v], rows_v, sem).wait()
        pltpu.sync_copy(rows_v, out_hbm.at[pl.ds(base, bpw)])

    return fused_gather


def kernel(x, lengths):
    B, T, D = x.shape
    return _build(B, T, D)(x.reshape(B * T, D), lengths)


# speedup vs baseline: 2.2788x; 2.2788x over previous
import functools

import jax
import jax.numpy as jnp
from jax import lax
from jax.experimental import pallas as pl
from jax.experimental.pallas import tpu as pltpu
from jax.experimental.pallas import tpu_sc as plsc


@functools.lru_cache(maxsize=None)
def _build(B, T, D):
    info = plsc.get_sparse_core_info()
    NC, NS = info.num_cores, info.num_subcores
    NW = NC * NS
    assert B % (8 * NW) == 0
    bpw = B // NW

    mesh = plsc.VectorSubcoreMesh(core_axis_name="c", subcore_axis_name="s")

    @functools.partial(
        pl.kernel,
        mesh=mesh,
        out_type=jax.ShapeDtypeStruct((B, D), jnp.float32),
        scratch_types=[
            pltpu.VMEM((bpw,), jnp.int32),
            pltpu.VMEM((bpw, D), jnp.float32),
            pltpu.SemaphoreType.DMA,
        ],
    )
    def fused_gather(x_hbm, len_hbm, out_hbm, len_v, rows_v, sem):
        wid = lax.axis_index("s") * NC + lax.axis_index("c")
        base = wid * bpw
        pltpu.sync_copy(len_hbm.at[pl.ds(base, bpw)], len_v)
        copies = []
        for i in range(bpw // 16):
            ln = len_v[pl.ds(i * 16, 16)]
            tv = jnp.where(ln == jnp.int32(0), jnp.int32(T - 1), ln - 1)
            for j in range(16):
                r = i * 16 + j
                cp = pltpu.make_async_copy(
                    x_hbm.at[base + r, tv[j]], rows_v.at[r], sem)
                cp.start()
                copies.append(cp)
        for cp in copies:
            cp.wait()
        pltpu.sync_copy(rows_v, out_hbm.at[pl.ds(base, bpw)])

    return fused_gather


def kernel(x, lengths):
    B, T, D = x.shape
    return _build(B, T, D)(x, lengths)


# speedup vs baseline: 2.2789x; 1.0001x over previous
import functools

import jax
import jax.numpy as jnp
from jax import lax
from jax.experimental import pallas as pl
from jax.experimental.pallas import tpu as pltpu
from jax.experimental.pallas import tpu_sc as plsc


@functools.lru_cache(maxsize=None)
def _build(B, T, D):
    info = plsc.get_sparse_core_info()
    NC, NS = info.num_cores, info.num_subcores
    NW = NC * NS
    assert B % (8 * NW) == 0
    bpw = B // NW

    mesh = plsc.VectorSubcoreMesh(core_axis_name="c", subcore_axis_name="s")

    @functools.partial(
        pl.kernel,
        mesh=mesh,
        out_type=jax.ShapeDtypeStruct((B, D), jnp.float32),
        scratch_types=[
            pltpu.VMEM((bpw,), jnp.int32),
            pltpu.VMEM((bpw, D), jnp.float32),
            pltpu.SemaphoreType.DMA,
        ],
        compiler_params=pltpu.CompilerParams(use_tc_tiling_on_sc=True),
    )
    def fused_gather(x_hbm, len_hbm, out_hbm, len_v, rows_v, sem):
        wid = lax.axis_index("s") * NC + lax.axis_index("c")
        base = wid * bpw
        pltpu.sync_copy(len_hbm.at[pl.ds(base, bpw)], len_v)
        copies = []
        for i in range(bpw // 16):
            ln = len_v[pl.ds(i * 16, 16)]
            tv = jnp.where(ln == jnp.int32(0), jnp.int32(T - 1), ln - 1)
            for j in range(16):
                r = i * 16 + j
                cp = pltpu.make_async_copy(
                    x_hbm.at[base + r, tv[j]], rows_v.at[r], sem)
                cp.start()
                copies.append(cp)
        for cp in copies:
            cp.wait()
        pltpu.sync_copy(rows_v, out_hbm.at[pl.ds(base, bpw)])

    return fused_gather


def kernel(x, lengths):
    B, T, D = x.shape
    return _build(B, T, D)(x, lengths)


# speedup vs baseline: 9.2948x; 4.0786x over previous
import functools

import jax
import jax.numpy as jnp
from jax import lax
from jax.experimental import pallas as pl
from jax.experimental.pallas import tpu as pltpu
from jax.experimental.pallas import tpu_sc as plsc


@functools.lru_cache(maxsize=None)
def _build(B, T, D):
    info = plsc.get_sparse_core_info()
    NC, NS = info.num_cores, info.num_subcores
    NW = NC * NS
    assert B % (8 * NW) == 0
    bpw = B // NW

    mesh = plsc.VectorSubcoreMesh(core_axis_name="c", subcore_axis_name="s")

    @functools.partial(
        pl.kernel,
        mesh=mesh,
        out_type=jax.ShapeDtypeStruct((B, D), jnp.float32),
        scratch_types=[
            pltpu.VMEM((bpw,), jnp.int32),
            pltpu.VMEM((bpw,), jnp.int32),
            pltpu.VMEM((bpw, D), jnp.float32),
            pltpu.SemaphoreType.DMA,
            pltpu.SemaphoreType.DMA,
            pltpu.SemaphoreType.DMA,
            pltpu.SemaphoreType.DMA,
            pltpu.SemaphoreType.DMA,
            pltpu.SemaphoreType.DMA,
        ],
        compiler_params=pltpu.CompilerParams(use_tc_tiling_on_sc=True),
    )
    def fused_gather(xf_hbm, len_hbm, out_hbm, len_v, idx_v, rows_v,
                     sem_l0, sem_l1, sem_g0, sem_g1, sem_w0, sem_w1):
        wid = lax.axis_index("s") * NC + lax.axis_index("c")
        base = wid * bpw
        half = bpw // 2
        lane = lax.iota(jnp.int32, 16)

        l0 = pltpu.make_async_copy(
            len_hbm.at[pl.ds(base, half)], len_v.at[pl.ds(0, half)], sem_l0)
        l0.start()
        l1 = pltpu.make_async_copy(
            len_hbm.at[pl.ds(base + half, half)],
            len_v.at[pl.ds(half, half)], sem_l1)
        l1.start()

        def fill_idx(lo, hi):
            for i in range(lo, hi):
                ln = len_v[pl.ds(i * 16, 16)]
                tv = jnp.where(ln == jnp.int32(0), jnp.int32(T - 1), ln - 1)
                idx_v[pl.ds(i * 16, 16)] = tv * B + (base + i * 16) + lane

        l0.wait()
        fill_idx(0, half // 16)
        g0 = pltpu.make_async_copy(
            xf_hbm.at[idx_v.at[pl.ds(0, half)]], rows_v.at[pl.ds(0, half)],
            sem_g0)
        g0.start()
        l1.wait()
        fill_idx(half // 16, bpw // 16)
        g1 = pltpu.make_async_copy(
            xf_hbm.at[idx_v.at[pl.ds(half, half)]],
            rows_v.at[pl.ds(half, half)], sem_g1)
        g1.start()
        g0.wait()
        w0 = pltpu.make_async_copy(
            rows_v.at[pl.ds(0, half)], out_hbm.at[pl.ds(base, half)], sem_w0)
        w0.start()
        g1.wait()
        w1 = pltpu.make_async_copy(
            rows_v.at[pl.ds(half, half)],
            out_hbm.at[pl.ds(base + half, half)], sem_w1)
        w1.start()
        w0.wait()
        w1.wait()

    return fused_gather


def kernel(x, lengths):
    B, T, D = x.shape
    x_t = jnp.transpose(x, (1, 0, 2))
    return _build(B, T, D)(x_t.reshape(T * B, D), lengths)
